# baseline (speedup 1.0000x reference)
import jax
import jax.numpy as jnp
from jax.experimental import pallas as pl
from jax.experimental.pallas import tpu as pltpu

NSEL = 384
H = 768
NPAD = 1024
TB = 128


def _body(xt_ref, w_ref, b_ref, o_ref):
    xt = xt_ref[...]
    w = w_ref[...]
    s = jax.lax.dot_general(w, xt, (((1,), (0,)), ((), ())))
    imp = jax.nn.sigmoid(s + b_ref[...])
    tb = xt.shape[1]
    k = pltpu.bitcast(imp, jnp.int32)
    k = jnp.concatenate([k, jnp.zeros((NPAD - H, tb), jnp.int32)], axis=0)
    i = jax.lax.broadcasted_iota(jnp.int32, (NPAD, tb), 0)
    d = jnp.concatenate([xt, jnp.zeros((NPAD - H, tb), jnp.float32)], axis=0)
    pos = jax.lax.broadcasted_iota(jnp.int32, (NPAD, tb), 0)

    blk = 2
    while blk <= NPAD:
        st = blk // 2
        while st >= 1:
            lower = (pos & st) != 0
            desc = (pos & blk) == 0
            kp = jnp.where(lower, jnp.roll(k, st, axis=0),
                           jnp.roll(k, -st, axis=0))
            ip = jnp.where(lower, jnp.roll(i, st, axis=0),
                           jnp.roll(i, -st, axis=0))
            dp = jnp.where(lower, jnp.roll(d, st, axis=0),
                           jnp.roll(d, -st, axis=0))
            self_first = (k > kp) | ((k == kp) & (i < ip))
            slot_first = jnp.logical_not(lower) == desc
            keep = self_first == slot_first
            k = jnp.where(keep, k, kp)
            i = jnp.where(keep, i, ip)
            d = jnp.where(keep, d, dp)
            st //= 2
        blk *= 2
    o_ref[...] = d[:NSEL, :]


def kernel(data, W, b):
    B, S, Hd = data.shape
    N = B * S
    xt = data.reshape(N, Hd).T
    b2 = b.reshape(Hd, 1)
    out_t = pl.pallas_call(
        _body,
        grid=(N // TB,),
        in_specs=[
            pl.BlockSpec((Hd, TB), lambda i: (0, i)),
            pl.BlockSpec((Hd, Hd), lambda i: (0, 0)),
            pl.BlockSpec((Hd, 1), lambda i: (0, 0)),
        ],
        out_specs=pl.BlockSpec((NSEL, TB), lambda i: (0, i)),
        out_shape=jax.ShapeDtypeStruct((NSEL, N), jnp.float32),
    )(xt, W, b2)
    return out_t.T.reshape(B, S, NSEL)


# speedup vs baseline: 2.0011x; 2.0011x over previous
import jax
import jax.numpy as jnp
from jax.experimental import pallas as pl
from jax.experimental.pallas import tpu as pltpu

NSEL = 384
H = 768
NPAD = 1024
TB = 128
NR = NPAD // 8


def _lex_first(ka, ia, kb, ib):
    return (ka > kb) | ((ka == kb) & (ia < ib))


def _body(xt_ref, w_ref, b_ref, o_ref):
    xt = xt_ref[...]
    w = w_ref[...]
    s = jax.lax.dot_general(w, xt, (((1,), (0,)), ((), ())))
    imp = jax.nn.sigmoid(s + b_ref[...])
    kk = pltpu.bitcast(imp, jnp.int32)
    k = jnp.concatenate(
        [kk, jnp.zeros((NPAD - H, TB), jnp.int32)], axis=0).reshape(NR, 8, TB)
    d = jnp.concatenate(
        [xt, jnp.zeros((NPAD - H, TB), jnp.float32)], axis=0).reshape(NR, 8, TB)
    rpos = jax.lax.broadcasted_iota(jnp.int32, (NR, 8, TB), 0)
    spos = jax.lax.broadcasted_iota(jnp.int32, (NR, 8, TB), 1)
    pos = rpos * 8 + spos
    i = pos

    blk = 2
    while blk <= NPAD:
        st = blk // 2
        while st >= 1:
            if st < 8:
                lower = (spos & st) != 0
                desc = (pos & blk) == 0
                kp = jnp.where(lower, jnp.roll(k, st, axis=1),
                               jnp.roll(k, -st, axis=1))
                ip = jnp.where(lower, jnp.roll(i, st, axis=1),
                               jnp.roll(i, -st, axis=1))
                dp = jnp.where(lower, jnp.roll(d, st, axis=1),
                               jnp.roll(d, -st, axis=1))
                self_first = _lex_first(k, i, kp, ip)
                slot_first = jnp.logical_not(lower) == desc
                keep = self_first == slot_first
                k = jnp.where(keep, k, kp)
                i = jnp.where(keep, i, ip)
                d = jnp.where(keep, d, dp)
            else:
                s8 = st // 8
                g = NR // (2 * s8)
                k5 = k.reshape(g, 2, s8, 8, TB)
                i5 = i.reshape(g, 2, s8, 8, TB)
                d5 = d.reshape(g, 2, s8, 8, TB)
                desc5 = ((pos & blk) == 0).reshape(g, 2, s8, 8, TB)
                ka, kb = k5[:, 0], k5[:, 1]
                ia, ib = i5[:, 0], i5[:, 1]
                da, db = d5[:, 0], d5[:, 1]
                desc_a = desc5[:, 0]
                a_first = _lex_first(ka, ia, kb, ib)
                a_stays = a_first == desc_a
                ka2 = jnp.where(a_stays, ka, kb)
                kb2 = jnp.where(a_stays, kb, ka)
                ia2 = jnp.where(a_stays, ia, ib)
                ib2 = jnp.where(a_stays, ib, ia)
                da2 = jnp.where(a_stays, da, db)
                db2 = jnp.where(a_stays, db, da)
                k = jnp.stack((ka2, kb2), axis=1).reshape(NR, 8, TB)
                i = jnp.stack((ia2, ib2), axis=1).reshape(NR, 8, TB)
                d = jnp.stack((da2, db2), axis=1).reshape(NR, 8, TB)
            st //= 2
        blk *= 2
    o_ref[...] = d.reshape(NPAD, TB)[:NSEL, :]


def kernel(data, W, b):
    B, S, Hd = data.shape
    N = B * S
    xt = data.reshape(N, Hd).T
    b2 = b.reshape(Hd, 1)
    out_t = pl.pallas_call(
        _body,
        grid=(N // TB,),
        in_specs=[
            pl.BlockSpec((Hd, TB), lambda i: (0, i)),
            pl.BlockSpec((Hd, Hd), lambda i: (0, 0)),
            pl.BlockSpec((Hd, 1), lambda i: (0, 0)),
        ],
        out_specs=pl.BlockSpec((NSEL, TB), lambda i: (0, i)),
        out_shape=jax.ShapeDtypeStruct((NSEL, N), jnp.float32),
    )(xt, W, b2)
    return out_t.T.reshape(B, S, NSEL)


# speedup vs baseline: 2.6819x; 1.3402x over previous
import jax
import jax.numpy as jnp
from jax.experimental import pallas as pl
from jax.experimental.pallas import tpu as pltpu

NSEL = 384
H = 768
NPAD = 1024
TB = 128


def _lex_first(ka, ia, kb, ib):
    return (ka > kb) | ((ka == kb) & (ia < ib))


def _stage(k, i, d, blk, st, drop_b=False):
    rows = k.shape[0]
    if st < 8:
        spos = jax.lax.broadcasted_iota(jnp.int32, (rows, 8, TB), 1)
        rpos = jax.lax.broadcasted_iota(jnp.int32, (rows, 8, TB), 0)
        pos = rpos * 8 + spos
        lower = (spos & st) != 0
        desc = (pos & blk) == 0
        kp = jnp.where(lower, jnp.roll(k, st, axis=1),
                       jnp.roll(k, -st, axis=1))
        ip = jnp.where(lower, jnp.roll(i, st, axis=1),
                       jnp.roll(i, -st, axis=1))
        dp = jnp.where(lower, jnp.roll(d, st, axis=1),
                       jnp.roll(d, -st, axis=1))
        self_first = _lex_first(k, i, kp, ip)
        slot_first = jnp.logical_not(lower) == desc
        keep = self_first == slot_first
        return (jnp.where(keep, k, kp), jnp.where(keep, i, ip),
                jnp.where(keep, d, dp))
    s8 = st // 8
    g = rows // (2 * s8)
    k5 = k.reshape(g, 2, s8, 8, TB)
    i5 = i.reshape(g, 2, s8, 8, TB)
    d5 = d.reshape(g, 2, s8, 8, TB)
    ka, kb = k5[:, 0], k5[:, 1]
    ia, ib = i5[:, 0], i5[:, 1]
    da, db = d5[:, 0], d5[:, 1]
    gpos = jax.lax.broadcasted_iota(jnp.int32, (g, s8, 8, TB), 0)
    desc_a = ((gpos * 2 * st) & blk) == 0
    a_first = _lex_first(ka, ia, kb, ib)
    a_stays = a_first == desc_a
    ka2 = jnp.where(a_stays, ka, kb)
    ia2 = jnp.where(a_stays, ia, ib)
    da2 = jnp.where(a_stays, da, db)
    if drop_b:
        return (ka2.reshape(rows // 2, 8, TB), ia2.reshape(rows // 2, 8, TB),
                da2.reshape(rows // 2, 8, TB))
    kb2 = jnp.where(a_stays, kb, ka)
    ib2 = jnp.where(a_stays, ib, ia)
    db2 = jnp.where(a_stays, db, da)
    return (jnp.stack((ka2, kb2), axis=1).reshape(rows, 8, TB),
            jnp.stack((ia2, ib2), axis=1).reshape(rows, 8, TB),
            jnp.stack((da2, db2), axis=1).reshape(rows, 8, TB))


def _sortnet(k, i, d):
    blk = 2
    while blk <= 256:
        st = blk // 2
        while st >= 1:
            k, i, d = _stage(k, i, d, blk, st)
            st //= 2
        blk *= 2

    zpad = jnp.zeros(((NPAD - H) // 8, 8, TB), jnp.int32)
    prpos = jax.lax.broadcasted_iota(jnp.int32, ((NPAD - H) // 8, 8, TB), 0)
    pspos = jax.lax.broadcasted_iota(jnp.int32, ((NPAD - H) // 8, 8, TB), 1)
    ipad = H + prpos * 8 + pspos
    k = jnp.concatenate([k, zpad], axis=0)
    i = jnp.concatenate([i, ipad], axis=0)
    d = jnp.concatenate(
        [d, jnp.zeros(((NPAD - H) // 8, 8, TB), jnp.float32)], axis=0)

    for st in (256, 128, 64, 32, 16, 8, 4, 2, 1):
        k, i, d = _stage(k, i, d, 512, st)

    k, i, d = _stage(k, i, d, 1024, 512, drop_b=True)
    k, i, d = _stage(k, i, d, 1024, 256)
    k, i, d = _stage(k, i, d, 1024, 128)
    k, i, d = k[:48], i[:48], d[:48]
    for st in (64, 32, 16, 8, 4, 2, 1):
        k, i, d = _stage(k, i, d, 1024, st)
    return d


def _body(xt_ref, w_ref, b_ref, o_ref):
    xt = xt_ref[...]
    w = w_ref[...]
    s = jax.lax.dot_general(w, xt, (((1,), (0,)), ((), ())))
    imp = jax.nn.sigmoid(s + b_ref[...])
    k = pltpu.bitcast(imp, jnp.int32).reshape(H // 8, 8, TB)
    d = xt.reshape(H // 8, 8, TB)
    rpos = jax.lax.broadcasted_iota(jnp.int32, (H // 8, 8, TB), 0)
    spos = jax.lax.broadcasted_iota(jnp.int32, (H // 8, 8, TB), 1)
    i = rpos * 8 + spos
    o_ref[...] = _sortnet(k, i, d).reshape(NSEL, TB)


def kernel(data, W, b):
    B, S, Hd = data.shape
    N = B * S
    xt = data.reshape(N, Hd).T
    b2 = b.reshape(Hd, 1)
    out_t = pl.pallas_call(
        _body,
        grid=(N // TB,),
        in_specs=[
            pl.BlockSpec((Hd, TB), lambda i: (0, i)),
            pl.BlockSpec((Hd, Hd), lambda i: (0, 0)),
            pl.BlockSpec((Hd, 1), lambda i: (0, 0)),
        ],
        out_specs=pl.BlockSpec((NSEL, TB), lambda i: (0, i)),
        out_shape=jax.ShapeDtypeStruct((NSEL, N), jnp.float32),
    )(xt, W, b2)
    return out_t.T.reshape(B, S, NSEL)
